# Initial kernel scaffold; baseline (speedup 1.0000x reference)
import jax
import jax.numpy as jnp
from jax.experimental import pallas as pl


def kernel(x, edge_index, W1, b1, W2, b2):
    raise NotImplementedError("write your pallas kernel here")


# speedup vs baseline: 36.1469x; 36.1469x over previous
import jax
import jax.numpy as jnp
from jax import lax
from jax.experimental import pallas as pl
from jax.experimental.pallas import tpu as pltpu
from jax.experimental.pallas import tpu_sc as plsc

_N = 10000
_E = 320000
_DH = 16
_DO = 20
_NS = 16
_BLK = 512
_NBLK = 40
_EPAD = _NS * _NBLK * _BLK
_N1 = 10240
_NSUB = _N1 // _NS
_CH = 2
_NPAIR = _NBLK // (2 * _CH)

_MESH = plsc.VectorSubcoreMesh(
    core_axis_name="c", subcore_axis_name="s",
    num_cores=1, num_subcores=_NS)

_SC_PARAMS = pltpu.CompilerParams(use_tc_tiling_on_sc=False)


def _deg_body(didx_hbm, ones_hbm, zmat_hbm, deg_hbm, didx_v, ones_v,
              deg_sh):
    s = lax.axis_index("s")
    pltpu.sync_copy(zmat_hbm, deg_sh.at[pl.ds(s * _NSUB, _NSUB)])
    pltpu.sync_copy(ones_hbm, ones_v)
    pltpu.sync_copy(didx_hbm.at[s], didx_v)
    plsc.subcore_barrier()

    def body(i, carry):
        for b in range(4):
            pltpu.sync_copy(ones_v, deg_sh.at[didx_v.at[i * 4 + b]],
                            add=True)
        return carry

    lax.fori_loop(0, _NBLK // 4, body, 0)
    plsc.subcore_barrier()
    pltpu.sync_copy(deg_sh.at[pl.ds(s * _NSUB, _NSUB)],
                    deg_hbm.at[pl.ds(s * _NSUB, _NSUB)])


_deg_call = pl.kernel(
    _deg_body,
    out_type=jax.ShapeDtypeStruct((_N1, _DH), jnp.float32),
    mesh=_MESH,
    compiler_params=_SC_PARAMS,
    scratch_types=[
        pltpu.VMEM((_NBLK, _BLK), jnp.int32),
        pltpu.VMEM((_BLK, _DH), jnp.float32),
        pltpu.MemorySpace.VMEM_SHARED((_N1, _DH), jnp.float32),
    ],
)


def _agg_body(g_hbm, sidx_hbm, didx_hbm, zmat_hbm, agg_hbm, sidx_v, didx_v,
              bufa, bufb, agg_sh, sema, semb):
    s = lax.axis_index("s")
    pltpu.sync_copy(zmat_hbm, agg_sh.at[pl.ds(s * _NSUB, _NSUB)])
    pltpu.sync_copy(sidx_hbm.at[s], sidx_v)
    pltpu.sync_copy(didx_hbm.at[s], didx_v)
    plsc.subcore_barrier()

    for b in range(_CH):
        pltpu.async_copy(g_hbm.at[sidx_v.at[b]], bufa.at[b], sema)

    def body(i, carry):
        base_a = (2 * i) * _CH
        base_b = base_a + _CH
        for b in range(_CH):
            pltpu.async_copy(g_hbm.at[sidx_v.at[base_b + b]], bufb.at[b],
                             semb)
        for b in range(_CH):
            pltpu.make_async_copy(g_hbm.at[sidx_v.at[base_a + b]],
                                  bufa.at[b], sema).wait()
        for b in range(_CH):
            pltpu.sync_copy(bufa.at[b], agg_sh.at[didx_v.at[base_a + b]],
                            add=True)

        @pl.when(i < _NPAIR - 1)
        def _():
            for b in range(_CH):
                pltpu.async_copy(g_hbm.at[sidx_v.at[base_b + _CH + b]],
                                 bufa.at[b], sema)

        for b in range(_CH):
            pltpu.make_async_copy(g_hbm.at[sidx_v.at[base_b + b]],
                                  bufb.at[b], semb).wait()
        for b in range(_CH):
            pltpu.sync_copy(bufb.at[b], agg_sh.at[didx_v.at[base_b + b]],
                            add=True)
        return carry

    lax.fori_loop(0, _NPAIR, body, 0)
    plsc.subcore_barrier()
    pltpu.sync_copy(agg_sh.at[pl.ds(s * _NSUB, _NSUB)],
                    agg_hbm.at[pl.ds(s * _NSUB, _NSUB)])


_agg_call = pl.kernel(
    _agg_body,
    out_type=jax.ShapeDtypeStruct((_N1, _DH), jnp.float32),
    mesh=_MESH,
    compiler_params=_SC_PARAMS,
    scratch_types=[
        pltpu.VMEM((_NBLK, _BLK), jnp.int32),
        pltpu.VMEM((_NBLK, _BLK), jnp.int32),
        pltpu.VMEM((_CH, _BLK, _DH), jnp.float32),
        pltpu.VMEM((_CH, _BLK, _DH), jnp.float32),
        pltpu.MemorySpace.VMEM_SHARED((_N1, _DH), jnp.float32),
        pltpu.SemaphoreType.DMA,
        pltpu.SemaphoreType.DMA,
    ],
)


def _tc_a_body(x_ref, w1_ref, degp_ref, g1_ref, dis_ref):
    deg = degp_ref[: _N, 0:1] + 1.0
    dis = lax.rsqrt(deg)
    h = jnp.dot(x_ref[...], w1_ref[...], preferred_element_type=jnp.float32,
                precision=lax.Precision.HIGHEST)
    g1_ref[...] = h * dis
    dis_ref[...] = dis


_tc_a = pl.pallas_call(
    _tc_a_body,
    out_shape=[
        jax.ShapeDtypeStruct((_N, _DH), jnp.float32),
        jax.ShapeDtypeStruct((_N, 1), jnp.float32),
    ],
)


def _tc_c_body(aggp_ref, g1_ref, dis_ref, b1_ref, z1s_ref):
    agg = aggp_ref[: _N, :] + g1_ref[...]
    dis = dis_ref[...]
    out1 = agg * dis + b1_ref[...]
    z1s_ref[...] = jnp.maximum(out1, 0.0) * dis


_tc_c = pl.pallas_call(
    _tc_c_body,
    out_shape=jax.ShapeDtypeStruct((_N, _DH), jnp.float32),
)


def _tc_e_body(aggp_ref, z1s_ref, dis_ref, w2_ref, b2_ref, out_ref):
    agg = aggp_ref[: _N, :] + z1s_ref[...]
    h = jnp.dot(agg, w2_ref[...], preferred_element_type=jnp.float32,
                precision=lax.Precision.HIGHEST)
    h = h * dis_ref[...] + b2_ref[...]
    m = jnp.max(h, axis=1, keepdims=True)
    lse = m + jnp.log(jnp.sum(jnp.exp(h - m), axis=1, keepdims=True))
    out_ref[...] = h - lse


_tc_e = pl.pallas_call(
    _tc_e_body,
    out_shape=jax.ShapeDtypeStruct((_N, _DO), jnp.float32),
)


def kernel(x, edge_index, W1, b1, W2, b2):
    src = edge_index[0]
    dst = edge_index[1]
    pad = _EPAD - _E
    sidx = jnp.concatenate(
        [src, jnp.zeros((pad,), jnp.int32)]).reshape(_NS, _NBLK, _BLK)
    didx = jnp.concatenate(
        [dst, jnp.full((pad,), _N, jnp.int32)]).reshape(_NS, _NBLK, _BLK)
    ones_blk = jnp.ones((_BLK, _DH), jnp.float32)
    zmat = jnp.zeros((_NSUB, _DH), jnp.float32)

    degp = _deg_call(didx, ones_blk, zmat)
    g1, dis = _tc_a(x, W1, degp)
    agg1p = _agg_call(g1, sidx, didx, zmat)
    z1s = _tc_c(agg1p, g1, dis, b1.reshape(1, _DH))
    agg2p = _agg_call(z1s, sidx, didx, zmat)
    return _tc_e(agg2p, z1s, dis, W2, b2.reshape(1, _DO))


# speedup vs baseline: 46.9642x; 1.2993x over previous
import jax
import jax.numpy as jnp
from jax import lax
from jax.experimental import pallas as pl
from jax.experimental.pallas import tpu as pltpu
from jax.experimental.pallas import tpu_sc as plsc

_N = 10000
_E = 320000
_DH = 16
_DO = 20
_NS = 16
_BLK = 1024
_NBLK = 20
_EPAD = _NS * _NBLK * _BLK
_DBLK = 2048
_DNBLK = _EPAD // (_NS * _DBLK)
_N1 = 10240
_NSUB = _N1 // _NS
_NST = _N // _NS
_CH = 2
_NPAIR = _NBLK // (2 * _CH)

_MESH = plsc.VectorSubcoreMesh(
    core_axis_name="c", subcore_axis_name="s",
    num_cores=1, num_subcores=_NS)

_SC_PARAMS = pltpu.CompilerParams(use_tc_tiling_on_sc=False)


def _deg_body(didx_hbm, ones_hbm, zmat_hbm, deg_hbm, didx_v, ones_v,
              deg_sh):
    s = lax.axis_index("s")
    pltpu.sync_copy(zmat_hbm, deg_sh.at[pl.ds(s * _NSUB, _NSUB)])
    pltpu.sync_copy(ones_hbm, ones_v)
    pltpu.sync_copy(didx_hbm.at[s], didx_v)
    plsc.subcore_barrier()

    def body(i, carry):
        pltpu.sync_copy(ones_v, deg_sh.at[didx_v.at[i]], add=True)
        return carry

    lax.fori_loop(0, _DNBLK, body, 0)
    plsc.subcore_barrier()
    pltpu.sync_copy(deg_sh.at[pl.ds(s * _NSUB, _NSUB)],
                    deg_hbm.at[pl.ds(s * _NSUB, _NSUB)])


_deg_call = pl.kernel(
    _deg_body,
    out_type=jax.ShapeDtypeStruct((_N1, _DH), jnp.float32),
    mesh=_MESH,
    compiler_params=_SC_PARAMS,
    scratch_types=[
        pltpu.VMEM((_DNBLK, _DBLK), jnp.int32),
        pltpu.VMEM((_DBLK, _DH), jnp.float32),
        pltpu.MemorySpace.VMEM_SHARED((_N1, _DH), jnp.float32),
    ],
)


def _agg_body(g_hbm, sidx_hbm, didx_hbm, zmat_hbm, agg_hbm, sidx_v, didx_v,
              bufa, bufb, agg_sh, gst_sh, sema, semb):
    s = lax.axis_index("s")
    pltpu.sync_copy(zmat_hbm, agg_sh.at[pl.ds(s * _NSUB, _NSUB)])
    pltpu.sync_copy(sidx_hbm.at[s], sidx_v)
    pltpu.sync_copy(didx_hbm.at[s], didx_v)
    pltpu.sync_copy(g_hbm.at[pl.ds(s * _NST, _NST)],
                    gst_sh.at[pl.ds(s * _NST, _NST)])
    plsc.subcore_barrier()

    for b in range(_CH):
        pltpu.async_copy(gst_sh.at[sidx_v.at[b]], bufa.at[b], sema)

    def body(i, carry):
        base_a = (2 * i) * _CH
        base_b = base_a + _CH
        for b in range(_CH):
            pltpu.async_copy(gst_sh.at[sidx_v.at[base_b + b]], bufb.at[b],
                             semb)
        for b in range(_CH):
            pltpu.make_async_copy(gst_sh.at[sidx_v.at[base_a + b]],
                                  bufa.at[b], sema).wait()
        for b in range(_CH):
            pltpu.sync_copy(bufa.at[b], agg_sh.at[didx_v.at[base_a + b]],
                            add=True)

        @pl.when(i < _NPAIR - 1)
        def _():
            for b in range(_CH):
                pltpu.async_copy(gst_sh.at[sidx_v.at[base_b + _CH + b]],
                                 bufa.at[b], sema)

        for b in range(_CH):
            pltpu.make_async_copy(gst_sh.at[sidx_v.at[base_b + b]],
                                  bufb.at[b], semb).wait()
        for b in range(_CH):
            pltpu.sync_copy(bufb.at[b], agg_sh.at[didx_v.at[base_b + b]],
                            add=True)
        return carry

    lax.fori_loop(0, _NPAIR, body, 0)
    plsc.subcore_barrier()
    pltpu.sync_copy(agg_sh.at[pl.ds(s * _NSUB, _NSUB)],
                    agg_hbm.at[pl.ds(s * _NSUB, _NSUB)])


_agg_call = pl.kernel(
    _agg_body,
    out_type=jax.ShapeDtypeStruct((_N1, _DH), jnp.float32),
    mesh=_MESH,
    compiler_params=_SC_PARAMS,
    scratch_types=[
        pltpu.VMEM((_NBLK, _BLK), jnp.int32),
        pltpu.VMEM((_NBLK, _BLK), jnp.int32),
        pltpu.VMEM((_CH, _BLK, _DH), jnp.float32),
        pltpu.VMEM((_CH, _BLK, _DH), jnp.float32),
        pltpu.MemorySpace.VMEM_SHARED((_N1, _DH), jnp.float32),
        pltpu.MemorySpace.VMEM_SHARED((_N, _DH), jnp.float32),
        pltpu.SemaphoreType.DMA,
        pltpu.SemaphoreType.DMA,
    ],
)


def _tc_a_body(x_ref, w1_ref, degp_ref, g1_ref, dis_ref):
    deg = degp_ref[: _N, 0:1] + 1.0
    dis = lax.rsqrt(deg)
    h = jnp.dot(x_ref[...], w1_ref[...], preferred_element_type=jnp.float32,
                precision=lax.Precision.HIGHEST)
    g1_ref[...] = h * dis
    dis_ref[...] = dis


_tc_a = pl.pallas_call(
    _tc_a_body,
    out_shape=[
        jax.ShapeDtypeStruct((_N, _DH), jnp.float32),
        jax.ShapeDtypeStruct((_N, 1), jnp.float32),
    ],
)


def _tc_c_body(aggp_ref, g1_ref, dis_ref, b1_ref, z1s_ref):
    agg = aggp_ref[: _N, :] + g1_ref[...]
    dis = dis_ref[...]
    out1 = agg * dis + b1_ref[...]
    z1s_ref[...] = jnp.maximum(out1, 0.0) * dis


_tc_c = pl.pallas_call(
    _tc_c_body,
    out_shape=jax.ShapeDtypeStruct((_N, _DH), jnp.float32),
)


def _tc_e_body(aggp_ref, z1s_ref, dis_ref, w2_ref, b2_ref, out_ref):
    agg = aggp_ref[: _N, :] + z1s_ref[...]
    h = jnp.dot(agg, w2_ref[...], preferred_element_type=jnp.float32,
                precision=lax.Precision.HIGHEST)
    h = h * dis_ref[...] + b2_ref[...]
    m = jnp.max(h, axis=1, keepdims=True)
    lse = m + jnp.log(jnp.sum(jnp.exp(h - m), axis=1, keepdims=True))
    out_ref[...] = h - lse


_tc_e = pl.pallas_call(
    _tc_e_body,
    out_shape=jax.ShapeDtypeStruct((_N, _DO), jnp.float32),
)


def kernel(x, edge_index, W1, b1, W2, b2):
    src = edge_index[0]
    dst = edge_index[1]
    pad = _EPAD - _E
    src_p = jnp.concatenate([src, jnp.zeros((pad,), jnp.int32)])
    dst_p = jnp.concatenate([dst, jnp.full((pad,), _N, jnp.int32)])
    sidx = src_p.reshape(_NS, _NBLK, _BLK)
    didx = dst_p.reshape(_NS, _NBLK, _BLK)
    didx_deg = dst_p.reshape(_NS, _DNBLK, _DBLK)
    ones_blk = jnp.ones((_DBLK, _DH), jnp.float32)
    zmat = jnp.zeros((_NSUB, _DH), jnp.float32)

    degp = _deg_call(didx_deg, ones_blk, zmat)
    g1, dis = _tc_a(x, W1, degp)
    agg1p = _agg_call(g1, sidx, didx, zmat)
    z1s = _tc_c(agg1p, g1, dis, b1.reshape(1, _DH))
    agg2p = _agg_call(z1s, sidx, didx, zmat)
    return _tc_e(agg2p, z1s, dis, W2, b2.reshape(1, _DO))


# speedup vs baseline: 49.0795x; 1.0450x over previous
import jax
import jax.numpy as jnp
from jax import lax
from jax.experimental import pallas as pl
from jax.experimental.pallas import tpu as pltpu
from jax.experimental.pallas import tpu_sc as plsc

_N = 10000
_E = 320000
_DH = 16
_DO = 20
_NS = 16
_BLK = 1024
_NBLK = 20
_EPAD = _NS * _NBLK * _BLK
_DBLK = 2048
_DNBLK = _EPAD // (_NS * _DBLK)
_N1 = 10240
_NSUB = _N1 // _NS
_CH = 2
_NPAIR = _NBLK // (2 * _CH)

_MESH = plsc.VectorSubcoreMesh(
    core_axis_name="c", subcore_axis_name="s",
    num_cores=1, num_subcores=_NS)

_SC_PARAMS = pltpu.CompilerParams(use_tc_tiling_on_sc=False)


def _deg_body(didx_hbm, ones_hbm, zmat_hbm, deg_hbm, didx_v, ones_v,
              deg_sh):
    s = lax.axis_index("s")
    pltpu.sync_copy(zmat_hbm, deg_sh.at[pl.ds(s * _NSUB, _NSUB)])
    pltpu.sync_copy(ones_hbm, ones_v)
    pltpu.sync_copy(didx_hbm.at[s], didx_v)
    plsc.subcore_barrier()

    def body(i, carry):
        pltpu.sync_copy(ones_v, deg_sh.at[didx_v.at[i]], add=True)
        return carry

    lax.fori_loop(0, _DNBLK, body, 0)
    plsc.subcore_barrier()
    pltpu.sync_copy(deg_sh.at[pl.ds(s * _NSUB, _NSUB)],
                    deg_hbm.at[pl.ds(s * _NSUB, _NSUB)])


_deg_call = pl.kernel(
    _deg_body,
    out_type=jax.ShapeDtypeStruct((_N1, _DH), jnp.float32),
    mesh=_MESH,
    compiler_params=_SC_PARAMS,
    scratch_types=[
        pltpu.VMEM((_DNBLK, _DBLK), jnp.int32),
        pltpu.VMEM((_DBLK, _DH), jnp.float32),
        pltpu.MemorySpace.VMEM_SHARED((_N1, _DH), jnp.float32),
    ],
)


def _fused_body(g1_hbm, sdidx_hbm, zmat_hbm, dism_hbm, b1_hbm,
                agg2_hbm, z1s_hbm, sidx_v, didx_v, bufa, bufb,
                hbuf, dmbuf, b1buf, agg_sh, gst_sh, sema, semb):
    s = lax.axis_index("s")
    row0 = s * _NSUB
    pltpu.sync_copy(zmat_hbm, agg_sh.at[pl.ds(row0, _NSUB)])
    pltpu.sync_copy(g1_hbm.at[pl.ds(row0, _NSUB)],
                    gst_sh.at[pl.ds(row0, _NSUB)])
    pltpu.sync_copy(sdidx_hbm.at[s], didx_v)
    pltpu.sync_copy(dism_hbm.at[pl.ds(row0, _NSUB)], dmbuf)
    pltpu.sync_copy(b1_hbm, b1buf)

    def unpack(i, carry):
        j = i // (_BLK // 16)
        k = (i % (_BLK // 16)) * 16
        w = didx_v[j, pl.ds(k, 16)]
        sidx_v[j, pl.ds(k, 16)] = jnp.bitwise_and(w, 0xFFFF)
        didx_v[j, pl.ds(k, 16)] = lax.shift_right_logical(w, 16)
        return carry

    lax.fori_loop(0, _NBLK * _BLK // 16, unpack, 0)
    plsc.subcore_barrier()

    def edge_sweep():
        pltpu.async_copy(gst_sh.at[sidx_v.at[0]], bufa, sema)

        def body(i, carry):
            ja = 2 * i
            jb = ja + 1
            pltpu.async_copy(gst_sh.at[sidx_v.at[jb]], bufb, semb)
            pltpu.make_async_copy(gst_sh.at[sidx_v.at[ja]], bufa, sema).wait()
            pltpu.sync_copy(bufa, agg_sh.at[didx_v.at[ja]], add=True)

            @pl.when(i < _NBLK // 2 - 1)
            def _():
                pltpu.async_copy(gst_sh.at[sidx_v.at[jb + 1]], bufa, sema)

            pltpu.make_async_copy(gst_sh.at[sidx_v.at[jb]], bufb, semb).wait()
            pltpu.sync_copy(bufb, agg_sh.at[didx_v.at[jb]], add=True)
            return carry

        lax.fori_loop(0, _NBLK // 2, body, 0)

    edge_sweep()
    plsc.subcore_barrier()

    abuf = bufa.at[pl.ds(0, _NSUB)]
    gbuf = bufb.at[pl.ds(0, _NSUB)]
    pltpu.sync_copy(agg_sh.at[pl.ds(row0, _NSUB)], abuf)
    pltpu.sync_copy(gst_sh.at[pl.ds(row0, _NSUB)], gbuf)
    b1v = b1buf[0, :]

    def rows(r, carry):
        a = abuf[r, :] + gbuf[r, :]
        d = dmbuf[r, :]
        z = jnp.maximum(a * d + b1v, 0.0) * d
        hbuf[r, :] = z
        return carry

    lax.fori_loop(0, _NSUB, rows, 0)
    pltpu.sync_copy(hbuf, gst_sh.at[pl.ds(row0, _NSUB)])
    pltpu.sync_copy(hbuf, z1s_hbm.at[pl.ds(row0, _NSUB)])
    pltpu.sync_copy(zmat_hbm, agg_sh.at[pl.ds(row0, _NSUB)])
    plsc.subcore_barrier()

    edge_sweep()
    plsc.subcore_barrier()
    pltpu.sync_copy(agg_sh.at[pl.ds(row0, _NSUB)],
                    agg2_hbm.at[pl.ds(row0, _NSUB)])


_fused_call = pl.kernel(
    _fused_body,
    out_type=[
        jax.ShapeDtypeStruct((_N1, _DH), jnp.float32),
        jax.ShapeDtypeStruct((_N1, _DH), jnp.float32),
    ],
    mesh=_MESH,
    compiler_params=_SC_PARAMS,
    scratch_types=[
        pltpu.VMEM((_NBLK, _BLK), jnp.int32),
        pltpu.VMEM((_NBLK, _BLK), jnp.int32),
        pltpu.VMEM((_BLK, _DH), jnp.float32),
        pltpu.VMEM((_BLK, _DH), jnp.float32),
        pltpu.VMEM((_NSUB, _DH), jnp.float32),
        pltpu.VMEM((_NSUB, _DH), jnp.float32),
        pltpu.VMEM((1, _DH), jnp.float32),
        pltpu.MemorySpace.VMEM_SHARED((_N1, _DH), jnp.float32),
        pltpu.MemorySpace.VMEM_SHARED((_N1, _DH), jnp.float32),
        pltpu.SemaphoreType.DMA,
        pltpu.SemaphoreType.DMA,
    ],
)


def _tc_a_body(x_ref, w1_ref, degp_ref, g1_ref, dism_ref, dis_ref):
    deg = degp_ref[:, 0:1] + 1.0
    dis_all = lax.rsqrt(deg)
    h = jnp.dot(x_ref[...], w1_ref[...], preferred_element_type=jnp.float32,
                precision=lax.Precision.HIGHEST)
    g1_ref[: _N, :] = h * dis_all[: _N, :]
    g1_ref[_N:, :] = jnp.zeros((_N1 - _N, _DH), jnp.float32)
    dism_ref[...] = dis_all * jnp.ones((1, _DH), jnp.float32)
    dis_ref[...] = dis_all[: _N, :]


_tc_a = pl.pallas_call(
    _tc_a_body,
    out_shape=[
        jax.ShapeDtypeStruct((_N1, _DH), jnp.float32),
        jax.ShapeDtypeStruct((_N1, _DH), jnp.float32),
        jax.ShapeDtypeStruct((_N, 1), jnp.float32),
    ],
)


def _tc_e_body(aggp_ref, z1s_ref, dis_ref, w2_ref, b2_ref, out_ref):
    agg = aggp_ref[: _N, :] + z1s_ref[: _N, :]
    h = jnp.dot(agg, w2_ref[...], preferred_element_type=jnp.float32,
                precision=lax.Precision.HIGHEST)
    h = h * dis_ref[...] + b2_ref[...]
    m = jnp.max(h, axis=1, keepdims=True)
    lse = m + jnp.log(jnp.sum(jnp.exp(h - m), axis=1, keepdims=True))
    out_ref[...] = h - lse


_tc_e = pl.pallas_call(
    _tc_e_body,
    out_shape=jax.ShapeDtypeStruct((_N, _DO), jnp.float32),
)


def kernel(x, edge_index, W1, b1, W2, b2):
    src = edge_index[0]
    dst = edge_index[1]
    pad = _EPAD - _E
    src_p = jnp.concatenate([src, jnp.zeros((pad,), jnp.int32)])
    dst_p = jnp.concatenate([dst, jnp.full((pad,), _N, jnp.int32)])
    sidx = src_p.reshape(_NS, _NBLK, _BLK)
    didx = dst_p.reshape(_NS, _NBLK, _BLK)
    didx_deg = dst_p.reshape(_NS, _DNBLK, _DBLK)
    ones_blk = jnp.ones((_DBLK, _DH), jnp.float32)
    zmat = jnp.zeros((_NSUB, _DH), jnp.float32)

    sdidx = (src_p | (dst_p << 16)).reshape(_NS, _NBLK, _BLK)
    degp = _deg_call(didx_deg, ones_blk, zmat)
    g1, dism, dis = _tc_a(x, W1, degp)
    agg2p, z1s = _fused_call(g1, sdidx, zmat, dism, b1.reshape(1, _DH))
    return _tc_e(agg2p, z1s, dis, W2, b2.reshape(1, _DO))


# speedup vs baseline: 50.4911x; 1.0288x over previous
import jax
import jax.numpy as jnp
from jax import lax
from jax.experimental import pallas as pl
from jax.experimental.pallas import tpu as pltpu
from jax.experimental.pallas import tpu_sc as plsc

_N = 10000
_E = 320000
_DH = 16
_DO = 20
_NS = 16
_BLK = 1024
_NBLK = 20
_EPAD = _NS * _NBLK * _BLK
_N1 = 10240
_NSUB = _N1 // _NS

_MESH = plsc.VectorSubcoreMesh(
    core_axis_name="c", subcore_axis_name="s",
    num_cores=1, num_subcores=_NS)

_SC_PARAMS = pltpu.CompilerParams(use_tc_tiling_on_sc=False,
                                  needs_layout_passes=False)

_RSQRT_MAGIC = 0x5F3759DF


def _mega_body(h1_hbm, sdidx_hbm, zmat_hbm, ones_hbm, b1_hbm,
               agg2_hbm, z1s_hbm, dism_hbm,
               sidx_v, didx_v, bufa, bufb, hbuf, dmbuf, b1buf,
               agg_sh, gst_sh, sema, semb):
    s = lax.axis_index("s")
    sub = pl.ds(s * _NSUB, _NSUB)
    pltpu.sync_copy(zmat_hbm, agg_sh.at[sub])
    pltpu.sync_copy(sdidx_hbm.at[s], didx_v)
    pltpu.sync_copy(ones_hbm, bufa)
    pltpu.sync_copy(b1_hbm, b1buf)

    def unpack(i, carry):
        j = i // (_BLK // 16)
        k = (i % (_BLK // 16)) * 16
        w = didx_v[j, pl.ds(k, 16)]
        sidx_v[j, pl.ds(k, 16)] = jnp.bitwise_and(w, 0xFFFF)
        didx_v[j, pl.ds(k, 16)] = lax.shift_right_logical(w, 16)
        return carry

    lax.fori_loop(0, _NBLK * _BLK // 16, unpack, 0)
    plsc.subcore_barrier()

    def deg_sweep(j, carry):
        pltpu.sync_copy(bufa, agg_sh.at[didx_v.at[j]], add=True)
        return carry

    lax.fori_loop(0, _NBLK, deg_sweep, 0)
    plsc.subcore_barrier()

    abuf = bufa.at[pl.ds(0, _NSUB)]
    gbuf = bufb.at[pl.ds(0, _NSUB)]
    pltpu.sync_copy(agg_sh.at[sub], gbuf)
    pltpu.sync_copy(h1_hbm.at[sub], abuf)

    def rows_dis(r, carry):
        deg = gbuf[r, :] + 1.0
        y = plsc.bitcast(deg, jnp.int32)
        y = jnp.full((_DH,), _RSQRT_MAGIC, jnp.int32) - (y >> 1)
        f = plsc.bitcast(y, jnp.float32)
        f = f * (1.5 - 0.5 * deg * f * f)
        f = f * (1.5 - 0.5 * deg * f * f)
        dmbuf[r, :] = f
        hbuf[r, :] = abuf[r, :] * f
        return carry

    lax.fori_loop(0, _NSUB, rows_dis, 0)
    pltpu.sync_copy(hbuf, gst_sh.at[sub])
    pltpu.sync_copy(dmbuf, dism_hbm.at[sub])
    pltpu.sync_copy(zmat_hbm, agg_sh.at[sub])
    plsc.subcore_barrier()

    def edge_sweep():
        pltpu.async_copy(gst_sh.at[sidx_v.at[0]], bufa, sema)

        def body(i, carry):
            ja = 2 * i
            jb = ja + 1
            pltpu.async_copy(gst_sh.at[sidx_v.at[jb]], bufb, semb)
            pltpu.make_async_copy(gst_sh.at[sidx_v.at[ja]], bufa, sema).wait()
            pltpu.sync_copy(bufa, agg_sh.at[didx_v.at[ja]], add=True)

            @pl.when(i < _NBLK // 2 - 1)
            def _():
                pltpu.async_copy(gst_sh.at[sidx_v.at[jb + 1]], bufa, sema)

            pltpu.make_async_copy(gst_sh.at[sidx_v.at[jb]], bufb, semb).wait()
            pltpu.sync_copy(bufb, agg_sh.at[didx_v.at[jb]], add=True)
            return carry

        lax.fori_loop(0, _NBLK // 2, body, 0)

    edge_sweep()
    plsc.subcore_barrier()

    pltpu.sync_copy(agg_sh.at[sub], abuf)
    b1v = b1buf[0, :]

    def rows_z(r, carry):
        a = abuf[r, :] + hbuf[r, :]
        d = dmbuf[r, :]
        hbuf[r, :] = jnp.maximum(a * d + b1v, 0.0) * d
        return carry

    lax.fori_loop(0, _NSUB, rows_z, 0)
    pltpu.sync_copy(hbuf, gst_sh.at[sub])
    pltpu.sync_copy(hbuf, z1s_hbm.at[sub])
    pltpu.sync_copy(zmat_hbm, agg_sh.at[sub])
    plsc.subcore_barrier()

    edge_sweep()
    plsc.subcore_barrier()
    pltpu.sync_copy(agg_sh.at[sub], agg2_hbm.at[sub])


_mega_call = pl.kernel(
    _mega_body,
    out_type=[
        jax.ShapeDtypeStruct((_N1, _DH), jnp.float32),
        jax.ShapeDtypeStruct((_N1, _DH), jnp.float32),
        jax.ShapeDtypeStruct((_N1, _DH), jnp.float32),
    ],
    mesh=_MESH,
    compiler_params=_SC_PARAMS,
    scratch_types=[
        pltpu.VMEM((_NBLK, _BLK), jnp.int32),
        pltpu.VMEM((_NBLK, _BLK), jnp.int32),
        pltpu.VMEM((_BLK, _DH), jnp.float32),
        pltpu.VMEM((_BLK, _DH), jnp.float32),
        pltpu.VMEM((_NSUB, _DH), jnp.float32),
        pltpu.VMEM((_NSUB, _DH), jnp.float32),
        pltpu.VMEM((1, _DH), jnp.float32),
        pltpu.MemorySpace.VMEM_SHARED((_N1, _DH), jnp.float32),
        pltpu.MemorySpace.VMEM_SHARED((_N1, _DH), jnp.float32),
        pltpu.SemaphoreType.DMA,
        pltpu.SemaphoreType.DMA,
    ],
)


def _tc_a_body(x_ref, w1_ref, h1_ref):
    h = jnp.dot(x_ref[...], w1_ref[...], preferred_element_type=jnp.float32,
                precision=lax.Precision.HIGHEST)
    h1_ref[: _N, :] = h
    h1_ref[_N:, :] = jnp.zeros((_N1 - _N, _DH), jnp.float32)


_tc_a = pl.pallas_call(
    _tc_a_body,
    out_shape=jax.ShapeDtypeStruct((_N1, _DH), jnp.float32),
)


def _tc_e_body(aggp_ref, z1s_ref, dism_ref, w2_ref, b2_ref, out_ref):
    agg = aggp_ref[: _N, :] + z1s_ref[: _N, :]
    h = jnp.dot(agg, w2_ref[...], preferred_element_type=jnp.float32,
                precision=lax.Precision.HIGHEST)
    h = h * dism_ref[: _N, 0:1] + b2_ref[...]
    m = jnp.max(h, axis=1, keepdims=True)
    lse = m + jnp.log(jnp.sum(jnp.exp(h - m), axis=1, keepdims=True))
    out_ref[...] = h - lse


_tc_e = pl.pallas_call(
    _tc_e_body,
    out_shape=jax.ShapeDtypeStruct((_N, _DO), jnp.float32),
)


def kernel(x, edge_index, W1, b1, W2, b2):
    src = edge_index[0]
    dst = edge_index[1]
    pad = _EPAD - _E
    src_p = jnp.concatenate([src, jnp.zeros((pad,), jnp.int32)])
    dst_p = jnp.concatenate([dst, jnp.full((pad,), _N, jnp.int32)])
    sdidx = (src_p | (dst_p << 16)).reshape(_NS, _NBLK, _BLK)
    ones_blk = jnp.ones((_BLK, _DH), jnp.float32)
    zmat = jnp.zeros((_NSUB, _DH), jnp.float32)

    h1 = _tc_a(x, W1)
    agg2p, z1s, dism = _mega_call(h1, sdidx, zmat, ones_blk,
                                  b1.reshape(1, _DH))
    return _tc_e(agg2p, z1s, dism, W2, b2.reshape(1, _DO))
